# baseline (speedup 1.0000x reference)
import functools

import jax
import jax.numpy as jnp
from jax import lax
from jax.experimental import pallas as pl
from jax.experimental.pallas import tpu as pltpu
from jax.experimental.pallas import tpu_sc as plsc

NUM_CLASS = 1000
D = 512
B = 16384
L = 16
NC, NS = 2, 16
NW = NC * NS
BPW = B // NW
CHUNK = 16
NSLOT = 4
NCHUNK = BPW // CHUNK
FCHUNK = D // L


def _body(x_hbm, labels_hbm, centers_hbm, out_hbm,
          idx_v, xbuf, cbuf, accbuf, out_v,
          *sems):
    wid = lax.axis_index("s") * NC + lax.axis_index("c")
    base = wid * BPW
    pltpu.sync_copy(labels_hbm.at[pl.ds(base, BPW)], idx_v)
    lane = lax.iota(jnp.int32, L)
    inv = jnp.float32(1.0 / D)
    lo = jnp.float32(1e-12)
    hi = jnp.float32(1e12)
    sems_x = sems[:NSLOT]
    sems_c = sems[NSLOT:]

    def start(k, slot):
        rb = k * CHUNK
        pltpu.async_copy(x_hbm.at[pl.ds(base + rb, CHUNK)],
                         xbuf.at[slot], sems_x[slot])
        pltpu.async_copy(centers_hbm.at[idx_v.at[pl.ds(rb, CHUNK)]],
                         cbuf.at[slot], sems_c[slot])

    def wait(slot):
        pltpu.make_async_copy(x_hbm.at[pl.ds(0, CHUNK)],
                              xbuf.at[slot], sems_x[slot]).wait()
        pltpu.make_async_copy(centers_hbm.at[idx_v.at[pl.ds(0, CHUNK)]],
                              cbuf.at[slot], sems_c[slot]).wait()

    def compute(slot, rb):
        def row_body(r, _):
            accs = [jnp.zeros((L,), jnp.float32) for _ in range(4)]
            for j in range(FCHUNK):
                xv = xbuf[slot, r, pl.ds(j * L, L)]
                cv = cbuf[slot, r, pl.ds(j * L, L)]
                d = xv - cv
                d2 = d * d
                d2 = jnp.minimum(jnp.maximum(d2, lo), hi)
                accs[j % 4] = accs[j % 4] + d2
            acc = (accs[0] + accs[1]) + (accs[2] + accs[3])
            accbuf[pl.ds(r * L, L)] = acc
            return 0
        lax.fori_loop(0, CHUNK, row_body, 0)

        for g in range(CHUNK // L):
            rows16 = (g * L + lane) * L
            tot = jnp.zeros((L,), jnp.float32)
            for c in range(L):
                tot = tot + plsc.load_gather(accbuf, [rows16 + c])
            out_v[pl.ds(rb + g * L, L)] = tot * inv

    for s in range(NSLOT):
        start(s, s)

    @pl.loop(0, NCHUNK // NSLOT)
    def ring(i):
        for s in range(NSLOT):
            k = i * NSLOT + s
            wait(s)
            compute(s, k * CHUNK)

            @pl.when(k + NSLOT < NCHUNK)
            def _():
                start(k + NSLOT, s)

    pltpu.sync_copy(out_v, out_hbm.at[pl.ds(base, BPW)])


@functools.partial(jax.jit, static_argnames=())
def kernel(x, labels, centers):
    labels = labels.astype(jnp.int32)
    mesh = plsc.VectorSubcoreMesh(core_axis_name="c", subcore_axis_name="s")
    fn = pl.kernel(
        _body,
        out_type=jax.ShapeDtypeStruct((B,), jnp.float32),
        mesh=mesh,
        scratch_types=[
            pltpu.VMEM((BPW,), jnp.int32),
            pltpu.VMEM((NSLOT, CHUNK, D), jnp.float32),
            pltpu.VMEM((NSLOT, CHUNK, D), jnp.float32),
            pltpu.VMEM((CHUNK * L,), jnp.float32),
            pltpu.VMEM((BPW,), jnp.float32),
        ] + [pltpu.SemaphoreType.DMA] * (2 * NSLOT),
        compiler_params=pltpu.CompilerParams(needs_layout_passes=False),
    )
    return fn(x, labels, centers)


# speedup vs baseline: 1.1327x; 1.1327x over previous
import functools

import jax
import jax.numpy as jnp
from jax import lax
from jax.experimental import pallas as pl
from jax.experimental.pallas import tpu as pltpu
from jax.experimental.pallas import tpu_sc as plsc

NUM_CLASS = 1000
D = 512
B = 16384
L = 16
NC, NS = 2, 16
NW = NC * NS
BPW = B // NW
CHUNK = 16
NSLOT = 4
NCHUNK = BPW // CHUNK
FCHUNK = D // L


def _body(x_hbm, labels_hbm, centers_hbm, out_hbm,
          idx_v, xbuf, cbuf, accbuf, out_v,
          *sems):
    wid = lax.axis_index("s") * NC + lax.axis_index("c")
    base = wid * BPW
    pltpu.sync_copy(labels_hbm.at[pl.ds(base, BPW)], idx_v)
    lane = lax.iota(jnp.int32, L)
    inv = jnp.float32(1.0 / D)
    lo = jnp.float32(1e-12)
    hi = jnp.float32(1e12)
    sems_x = sems[:NSLOT]
    sems_c = sems[NSLOT:]

    def start(k, slot):
        rb = k * CHUNK
        pltpu.async_copy(x_hbm.at[pl.ds(base + rb, CHUNK)],
                         xbuf.at[slot], sems_x[slot])
        pltpu.async_copy(centers_hbm.at[idx_v.at[pl.ds(rb, CHUNK)]],
                         cbuf.at[slot], sems_c[slot])

    def wait(slot):
        pltpu.make_async_copy(x_hbm.at[pl.ds(0, CHUNK)],
                              xbuf.at[slot], sems_x[slot]).wait()
        pltpu.make_async_copy(centers_hbm.at[idx_v.at[pl.ds(0, CHUNK)]],
                              cbuf.at[slot], sems_c[slot]).wait()

    def compute(slot, rb):
        @plsc.parallel_loop(0, CHUNK, unroll=2)
        def row_body(r):
            accs = [jnp.zeros((L,), jnp.float32) for _ in range(4)]
            for j in range(FCHUNK):
                xv = xbuf[slot, r, pl.ds(j * L, L)]
                cv = cbuf[slot, r, pl.ds(j * L, L)]
                d = xv - cv
                d2 = d * d
                d2 = jnp.minimum(jnp.maximum(d2, lo), hi)
                accs[j % 4] = accs[j % 4] + d2
            acc = (accs[0] + accs[1]) + (accs[2] + accs[3])
            accbuf[pl.ds(r * L, L)] = acc

        for g in range(CHUNK // L):
            rows16 = (g * L + lane) * L
            tot = jnp.zeros((L,), jnp.float32)
            for c in range(L):
                tot = tot + plsc.load_gather(accbuf, [rows16 + c])
            out_v[pl.ds(rb + g * L, L)] = tot * inv

    for s in range(NSLOT):
        start(s, s)

    @pl.loop(0, NCHUNK // NSLOT)
    def ring(i):
        for s in range(NSLOT):
            k = i * NSLOT + s
            wait(s)
            compute(s, k * CHUNK)

            @pl.when(k + NSLOT < NCHUNK)
            def _():
                start(k + NSLOT, s)

    pltpu.sync_copy(out_v, out_hbm.at[pl.ds(base, BPW)])


@functools.partial(jax.jit, static_argnames=())
def kernel(x, labels, centers):
    labels = labels.astype(jnp.int32)
    mesh = plsc.VectorSubcoreMesh(core_axis_name="c", subcore_axis_name="s")
    fn = pl.kernel(
        _body,
        out_type=jax.ShapeDtypeStruct((B,), jnp.float32),
        mesh=mesh,
        scratch_types=[
            pltpu.VMEM((BPW,), jnp.int32),
            pltpu.VMEM((NSLOT, CHUNK, D), jnp.float32),
            pltpu.VMEM((NSLOT, CHUNK, D), jnp.float32),
            pltpu.VMEM((CHUNK * L,), jnp.float32),
            pltpu.VMEM((BPW,), jnp.float32),
        ] + [pltpu.SemaphoreType.DMA] * (2 * NSLOT),
        compiler_params=pltpu.CompilerParams(needs_layout_passes=False),
    )
    return fn(x, labels, centers)
